# baseline (speedup 1.0000x reference)
import jax
import jax.numpy as jnp
from jax.experimental import pallas as pl

HIDDEN = 2048
N_EXPERTS = 64
BLOCK_M = 1024


def _router_kernel(hs_ref, w_ref, out_ref):
    out_ref[...] = jax.lax.dot_general(
        hs_ref[...],
        w_ref[...],
        dimension_numbers=(((1,), (1,)), ((), ())),
        preferred_element_type=jnp.float32,
    )


def kernel(hidden_states, weight):
    hs = hidden_states.reshape(-1, HIDDEN).astype(jnp.float32)
    w = weight.astype(jnp.float32)
    m = hs.shape[0]
    return pl.pallas_call(
        _router_kernel,
        grid=(m // BLOCK_M,),
        in_specs=[
            pl.BlockSpec((BLOCK_M, HIDDEN), lambda i: (i, 0)),
            pl.BlockSpec((N_EXPERTS, HIDDEN), lambda i: (0, 0)),
        ],
        out_specs=pl.BlockSpec((BLOCK_M, N_EXPERTS), lambda i: (i, 0)),
        out_shape=jax.ShapeDtypeStruct((m, N_EXPERTS), jnp.float32),
    )(hs, w)
